# baseline (speedup 1.0000x reference)
import functools

import jax
import jax.numpy as jnp
from jax import lax
from jax.experimental import pallas as pl
from jax.experimental.pallas import tpu as pltpu
from jax.experimental.pallas import tpu_sc as plsc

N, D, S = 320000, 128, 10000
NC, NS, L = 2, 16, 16
NW = NC * NS
RW = N // NW
BLK = 80
NBLK = RW // BLK
NBUF = 4
SROWS = 624
TAIL_OFF = SROWS * NS
TAIL = S - TAIL_OFF

WBLK = 4000


WGRP = 8


def _weights_body(f_ref, w_ref, b_ref, o_ref):
    f = f_ref[...]
    logits = jnp.sum(f * w_ref[...][None], axis=2) + b_ref[0, 0]
    o_ref[...] = jax.nn.sigmoid(logits)


def _row_weights(feats, W, b):
    feats4 = feats.reshape(N // WBLK, WBLK, D)
    return pl.pallas_call(
        _weights_body,
        grid=(N // (WBLK * WGRP),),
        in_specs=[
            pl.BlockSpec((WGRP, WBLK, D), lambda i: (i, 0, 0)),
            pl.BlockSpec((1, D), lambda i: (0, 0)),
            pl.BlockSpec(memory_space=pltpu.SMEM),
        ],
        out_specs=pl.BlockSpec((WGRP, WBLK), lambda i: (i, 0)),
        out_shape=jax.ShapeDtypeStruct((N // WBLK, WBLK), jnp.float32),
    )(feats4, W, b)


def _sc_pool(feats, iw, zeros):
    mesh = plsc.VectorSubcoreMesh(
        core_axis_name="c", subcore_axis_name="s",
        num_cores=NC, num_subcores=NS)

    fb_t = pltpu.VMEM((BLK, D), jnp.float32)
    iw_t = pltpu.VMEM((2, BLK), jnp.int32)

    @functools.partial(
        pl.kernel,
        out_type=jax.ShapeDtypeStruct((NC, S, D), jnp.float32),
        mesh=mesh,
        compiler_params=pltpu.CompilerParams(
            use_tc_tiling_on_sc=False, needs_layout_passes=False),
        scratch_types=(
            [fb_t] * NBUF + [iw_t] * NBUF
            + [pltpu.VMEM_SHARED((S, D), jnp.float32)]
            + [pltpu.SemaphoreType.DMA] * NBUF
        ),
    )
    def k(feats_hbm, iw_hbm, z_hbm, out_hbm, *scratch):
        fbufs = scratch[:NBUF]
        iwbufs = scratch[NBUF:2 * NBUF]
        acc = scratch[2 * NBUF]
        sems = scratch[2 * NBUF + 1:]

        c = lax.axis_index("c")
        s = lax.axis_index("s")
        wid = c * NS + s
        base = wid * RW

        def start_in(i, p):
            r0 = pl.multiple_of(base + i * BLK, 8)
            pltpu.async_copy(feats_hbm.at[pl.ds(r0, BLK), :], fbufs[p], sems[p])
            pltpu.async_copy(iw_hbm.at[wid * NBLK + i], iwbufs[p], sems[p])

        def wait_in(i, p):
            r0 = pl.multiple_of(base + i * BLK, 8)
            pltpu.make_async_copy(
                feats_hbm.at[pl.ds(r0, BLK), :], fbufs[p], sems[p]).wait()
            pltpu.make_async_copy(
                iw_hbm.at[wid * NBLK + i], iwbufs[p], sems[p]).wait()

        def process(p):
            fb, iwb = fbufs[p], iwbufs[p]

            def grp_body(g, rc):
                wv = plsc.bitcast(iwb[1, pl.ds(g * L, L)], jnp.float32)
                for j in range(L):
                    r = g * L + j
                    ws = wv[j]
                    for kk in range(D // L):
                        sl = pl.ds(kk * L, L)
                        fb[r, sl] = fb[r, sl] * ws
                return rc
            lax.fori_loop(0, BLK // L, grp_body, 0)

        def scatter(p):
            pltpu.sync_copy(fbufs[p], acc.at[iwbufs[p].at[0]], add=True)

        pltpu.sync_copy(z_hbm.at[pl.ds(s * SROWS, SROWS), :],
                        acc.at[pl.ds(s * SROWS, SROWS), :])

        @pl.when(s == 0)
        def _zero_tail():
            pltpu.sync_copy(z_hbm.at[pl.ds(TAIL_OFF, TAIL), :],
                            acc.at[pl.ds(TAIL_OFF, TAIL), :])
        plsc.subcore_barrier()

        for p in range(NBUF - 1):
            start_in(p, p)

        def quad_body(ip, carry):
            i0 = ip * NBUF
            for j in range(NBUF):
                i = i0 + j
                wait_in(i, j)

                @pl.when(i + NBUF - 1 < NBLK)
                def _fire():
                    start_in(i + NBUF - 1, (j + NBUF - 1) % NBUF)
                process(j)
                scatter(j)
            return carry
        lax.fori_loop(0, NBLK // NBUF, quad_body, 0)

        wait_in(NBLK - 1, (NBLK - 1) % NBUF)
        process((NBLK - 1) % NBUF)
        scatter((NBLK - 1) % NBUF)

        plsc.subcore_barrier()
        pltpu.sync_copy(acc.at[pl.ds(s * SROWS, SROWS), :],
                        out_hbm.at[c, pl.ds(s * SROWS, SROWS), :])

        @pl.when(s == 0)
        def _drain_tail():
            pltpu.sync_copy(acc.at[pl.ds(TAIL_OFF, TAIL), :],
                            out_hbm.at[c, pl.ds(TAIL_OFF, TAIL), :])

    return k(feats, iw, zeros)


def _combine_body(p_ref, o_ref):
    o_ref[...] = p_ref[0] + p_ref[1]


def _combine(parts):
    CB = 1000
    return pl.pallas_call(
        _combine_body,
        grid=(S // CB,),
        in_specs=[pl.BlockSpec((NC, CB, D), lambda i: (0, i, 0))],
        out_specs=pl.BlockSpec((CB, D), lambda i: (i, 0)),
        out_shape=jax.ShapeDtypeStruct((S, D), jnp.float32),
    )(parts)


def kernel(feats, batch, W, b):
    w = _row_weights(feats, W.reshape(1, D), b.reshape(1, 1)).reshape(N)
    wbits = jax.lax.bitcast_convert_type(w, jnp.int32).reshape(N // BLK, BLK)
    iw = jnp.stack([batch.reshape(N // BLK, BLK), wbits], axis=1)
    parts = _sc_pool(feats, iw, jnp.zeros((S, D), jnp.float32))
    return _combine(parts)
